# Initial kernel scaffold; baseline (speedup 1.0000x reference)
import jax
import jax.numpy as jnp
from jax.experimental import pallas as pl


def kernel(edge_index, edge_values, emb_user, emb_fakers, emb_item, gc_w0, gc_b0, gc_w1, gc_b1, bi_w0, bi_b0, bi_w1, bi_b1):
    raise NotImplementedError("write your pallas kernel here")


# speedup vs baseline: 5.8259x; 5.8259x over previous
import functools

import jax
import jax.numpy as jnp
from jax import lax
from jax.experimental import pallas as pl
from jax.experimental.pallas import tpu as pltpu
from jax.experimental.pallas import tpu_sc as plsc

N = 10000
D = 128
E = 320000
NC = 2
NS = 16
NW = NC * NS
EPW = E // NW
CH = 80
NCHUNK = EPW // CH
RPT = N // NS


def _spmm_body(emb_hbm, src_hbm, dst_hbm, vals_hbm, out_hbm,
               src_v, dst_v, vals_v, rows_v, dstc_v, acc_sh, sem):
    cid = lax.axis_index("c")
    sid = lax.axis_index("s")
    wid = cid * NS + sid

    zeros16 = jnp.zeros((16,), jnp.float32)

    @pl.loop(0, CH)
    def _zero_rows(i):
        for j in range(D // 16):
            rows_v[i, pl.ds(j * 16, 16)] = zeros16

    @pl.loop(sid, N // CH, step=NS)
    def _zero_acc(cz):
        pltpu.sync_copy(rows_v, acc_sh.at[pl.ds(cz * CH, CH)])

    plsc.subcore_barrier()

    ebase = wid * EPW
    pltpu.sync_copy(src_hbm.at[pl.ds(ebase, EPW)], src_v)
    pltpu.sync_copy(dst_hbm.at[pl.ds(ebase, EPW)], dst_v)
    pltpu.sync_copy(vals_hbm.at[pl.ds(ebase, EPW)], vals_v)

    @pl.loop(0, NCHUNK)
    def _chunk(ch):
        eb = ch * CH
        pltpu.async_copy(emb_hbm.at[src_v.at[pl.ds(eb, CH)]], rows_v,
                         sem).wait()
        for m in range(CH // 16):
            dstc_v[pl.ds(m * 16, 16)] = dst_v[pl.ds(eb + m * 16, 16)]

        @pl.loop(0, CH // 16)
        def _scale(g):
            v16 = vals_v[pl.ds(eb + g * 16, 16)]
            for r in range(16):
                v = v16[r]
                row = g * 16 + r
                for j in range(D // 16):
                    rows_v[row, pl.ds(j * 16, 16)] = (
                        rows_v[row, pl.ds(j * 16, 16)] * v)

        pltpu.sync_copy(rows_v, acc_sh.at[dstc_v], add=True)

    plsc.subcore_barrier()

    @pl.loop(sid, N // CH, step=NS)
    def _writeout(co):
        r0 = co * CH
        pltpu.sync_copy(acc_sh.at[pl.ds(r0, CH)], rows_v)
        pltpu.sync_copy(rows_v, out_hbm.at[pl.ds(cid * N + r0, CH)])


_spmm_sc = pl.kernel(
    _spmm_body,
    out_type=jax.ShapeDtypeStruct((NC * N, D), jnp.float32),
    mesh=plsc.VectorSubcoreMesh(core_axis_name="c", subcore_axis_name="s"),
    scratch_types=[
        pltpu.VMEM((EPW,), jnp.int32),
        pltpu.VMEM((EPW,), jnp.int32),
        pltpu.VMEM((EPW,), jnp.float32),
        pltpu.VMEM((CH, D), jnp.float32),
        pltpu.VMEM((CH,), jnp.int32),
        pltpu.VMEM_SHARED((N, D), jnp.float32),
        pltpu.SemaphoreType.DMA,
    ],
)


def _dense_body(p_ref, wg_ref, bg_ref, wb_ref, bb_ref, emb_ref, mlp_ref):
    s = p_ref[0] + p_ref[1]
    h = jnp.dot(s, wg_ref[...], preferred_element_type=jnp.float32)
    h = h + bg_ref[...]
    h = jnp.where(h >= 0, h, 0.01 * h)
    emb_ref[...] = h
    mlp_ref[...] = (jnp.dot(h, wb_ref[...], preferred_element_type=jnp.float32)
                    + bb_ref[...])


_BN = 2000


@functools.partial(jax.jit, static_argnums=())
def _dense_tc(partials, wg_t, bg, wb_t, bb):
    grid = N // _BN
    return pl.pallas_call(
        _dense_body,
        grid=(grid,),
        in_specs=[
            pl.BlockSpec((2, _BN, D), lambda i: (0, i, 0)),
            pl.BlockSpec((D, D), lambda i: (0, 0)),
            pl.BlockSpec((1, D), lambda i: (0, 0)),
            pl.BlockSpec((D, D), lambda i: (0, 0)),
            pl.BlockSpec((1, D), lambda i: (0, 0)),
        ],
        out_specs=[
            pl.BlockSpec((_BN, D), lambda i: (i, 0)),
            pl.BlockSpec((_BN, D), lambda i: (i, 0)),
        ],
        out_shape=[
            jax.ShapeDtypeStruct((N, D), jnp.float32),
            jax.ShapeDtypeStruct((N, D), jnp.float32),
        ],
    )(partials, wg_t, bg, wb_t, bb)


def kernel(edge_index, edge_values, emb_user, emb_fakers, emb_item,
           gc_w0, gc_b0, gc_w1, gc_b1, bi_w0, bi_b0, bi_w1, bi_b1):
    emb0 = jnp.concatenate([emb_user, emb_fakers, emb_item], axis=0)
    dst = edge_index[0]
    src = edge_index[1]

    def layer(emb, wg, bg, wb, bb):
        partials = _spmm_sc(emb, src, dst, edge_values).reshape(NC, N, D)
        return _dense_tc(partials, wg.T, bg.reshape(1, D),
                         wb.T, bb.reshape(1, D))

    emb1, mlp1 = layer(emb0, gc_w0, gc_b0, bi_w0, bi_b0)
    emb2, mlp2 = layer(emb1, gc_w1, gc_b1, bi_w1, bi_b1)

    all_emb = jnp.concatenate([emb0, mlp1, mlp2], axis=1)
    n_users_total = 5200
    return (all_emb[:n_users_total], all_emb[n_users_total:])
